# baseline (speedup 1.0000x reference)
import functools

import jax
import jax.numpy as jnp
from jax import lax
from jax.experimental import pallas as pl
from jax.experimental.pallas import tpu as pltpu
from jax.experimental.pallas import tpu_sc as plsc

B = 16384
D = 16
NC = 2
NS = 16
NW = NC * NS
BPW = B // NW
CHUNK = 128
NCHUNK = BPW // CHUNK

LANES = 128
ROWS = B * D // LANES
GRID = 8
BLK = ROWS // GRID
GROUPS = LANES // D


def _sc_gather(users, movies, tab_u, tab_m, bu_t, bm_t):
  mesh = plsc.VectorSubcoreMesh(core_axis_name="c", subcore_axis_name="s")
  f32 = jnp.float32

  @functools.partial(
      pl.kernel,
      mesh=mesh,
      out_type=[
          jax.ShapeDtypeStruct((B * D,), f32),
          jax.ShapeDtypeStruct((B * D,), f32),
          jax.ShapeDtypeStruct((B * D,), f32),
          jax.ShapeDtypeStruct((B * D,), f32),
          jax.ShapeDtypeStruct((B,), f32),
          jax.ShapeDtypeStruct((B,), f32),
      ],
      scratch_types=[
          pltpu.VMEM((BPW,), jnp.int32),
          pltpu.VMEM((BPW,), jnp.int32),
          pltpu.VMEM((BPW * D,), f32),
          pltpu.VMEM((BPW * D,), f32),
          pltpu.VMEM((BPW * D,), f32),
          pltpu.VMEM((BPW * D,), f32),
          pltpu.VMEM((BPW,), f32),
          pltpu.VMEM((BPW,), f32),
          pltpu.SemaphoreType.DMA,
          pltpu.SemaphoreType.DMA,
      ],
  )
  def k(us_h, mv_h, tu_h, tm_h, bu_h, bm_h,
        au_o, buo_o, am_o, bmo_o, bug_o, bmg_o,
        us_v, mv_v, au_v, buv_v, am_v, bmv_v,
        bus_v, bms_v, isem, sem):
    wid = lax.axis_index("s") * NC + lax.axis_index("c")
    base = wid * BPW
    sl = pl.ds(base, BPW)

    hs = [pltpu.async_copy(us_h.at[sl], us_v, isem),
          pltpu.async_copy(mv_h.at[sl], mv_v, isem)]
    for h in hs:
      h.wait()

    bias_hs = []
    for tab, idx_v, dst_v in ((bu_h, us_v, bus_v), (bm_h, mv_v, bms_v)):
      for c in range(NCHUNK):
        cs = pl.ds(c * CHUNK, CHUNK)
        bias_hs.append(
            pltpu.async_copy(tab.at[idx_v.at[cs]], dst_v.at[cs], isem))

    def body(g, _):
      gsl = pl.ds(pl.multiple_of(g * 16, 16), 16)
      uvec = us_v[gsl]
      mvec = mv_v[gsl]
      for k in range(16):
        u = pl.multiple_of(uvec[k] * (2 * D), 2 * D)
        m = pl.multiple_of(mvec[k] * (2 * D), 2 * D)
        ub = pl.multiple_of(uvec[k] * (2 * D) + D, D)
        mb = pl.multiple_of(mvec[k] * (2 * D) + D, D)
        o = pl.multiple_of(g * (16 * D) + k * D, D)
        pltpu.async_copy(tu_h.at[pl.ds(u, D)], au_v.at[pl.ds(o, D)], sem)
        pltpu.async_copy(tu_h.at[pl.ds(ub, D)], buv_v.at[pl.ds(o, D)], sem)
        pltpu.async_copy(tm_h.at[pl.ds(m, D)], am_v.at[pl.ds(o, D)], sem)
        pltpu.async_copy(tm_h.at[pl.ds(mb, D)], bmv_v.at[pl.ds(o, D)], sem)
      return ()
    lax.fori_loop(0, BPW // 16, body, ())

    for buf in (au_v, buv_v, am_v, bmv_v):
      pltpu.make_async_copy(tu_h.at[pl.ds(0, BPW * D)], buf, sem).wait()
    for h in bias_hs:
      h.wait()

    fsl = pl.ds(base * D, BPW * D)
    outs = [(au_v, au_o.at[fsl]), (buv_v, buo_o.at[fsl]),
            (am_v, am_o.at[fsl]), (bmv_v, bmo_o.at[fsl]),
            (bus_v, bug_o.at[sl]), (bms_v, bmg_o.at[sl])]
    hs = [pltpu.async_copy(src, dst, isem) for src, dst in outs]
    for h in hs:
      h.wait()

  return k(users, movies, tab_u, tab_m, bu_t, bm_t)


_HL2PI = 0.9189385332046727


def _stirling(z):
  r = 1.0 / z
  w = r * r
  series = r * (8.333333333333333e-2
                + w * (-2.777777777777778e-3 + w * 7.936507936507937e-4))
  return (z - 0.5) * jnp.log(z) - z + _HL2PI + series


def _lgamma_1(x):
  return _stirling(x + 3.0) - jnp.log(x * (x + 1.0) * (x + 2.0))


def _lgamma_2(s):
  return _stirling(s + 2.0) - jnp.log(s * (s + 1.0))


def _dg_series(z):
  r = 1.0 / z
  w = r * r
  return (jnp.log(z) - 0.5 * r
          - w * (8.333333333333333e-2
                 + w * (-8.333333333333333e-3 + w * 3.968253968253968e-3)))


def _digamma_1(x):
  num = 3.0 * x * x + 6.0 * x + 2.0
  den = x * (x + 1.0) * (x + 2.0)
  return _dg_series(x + 3.0) - num / den

def _digamma_2(s):
  return _dg_series(s + 2.0) - (2.0 * s + 1.0) / (s * (s + 1.0))


def _math_body(au_ref, bu_ref, am_ref, bm_ref, bug_ref, bmg_ref, out_ref):
  def fix(v):
    v = jnp.where(jnp.isnan(v), 0.05, v)
    return jnp.clip(v + 1.0, 1.0, 100.0)

  a1 = fix(au_ref[...])
  b1 = fix(bu_ref[...])
  a2 = fix(am_ref[...])
  b2 = fix(bm_ref[...])
  s1 = a1 + b1
  s2 = a2 + b2

  lnB1 = _lgamma_1(a1) + _lgamma_1(b1) - _lgamma_2(s1)
  lnB2 = _lgamma_1(a2) + _lgamma_1(b2) - _lgamma_2(s2)
  kl = (lnB2 - lnB1
        + (a1 - a2) * _digamma_1(a1)
        + (b1 - b2) * _digamma_1(b1)
        + (a2 - a1 + b2 - b1) * _digamma_2(s1))

  t = jnp.arctan2(jnp.abs(kl), 1.0) * (2.0 / jnp.pi)

  ri = lax.broadcasted_iota(jnp.int32, (LANES, GROUPS), 0)
  ci = lax.broadcasted_iota(jnp.int32, (LANES, GROUPS), 1)
  sel = jnp.where((ri // D) == ci, 1.0, 0.0).astype(jnp.float32)
  dist = jnp.dot(t, sel, preferred_element_type=jnp.float32)

  out_ref[...] = bug_ref[...] + bmg_ref[...] - dist


def _tc_math(au2, bu2, am2, bm2, bug2, bmg2):
  wide = pl.BlockSpec((BLK, LANES), lambda i: (i, 0))
  slim = pl.BlockSpec((BLK, GROUPS), lambda i: (i, 0))
  return pl.pallas_call(
      _math_body,
      grid=(GRID,),
      in_specs=[wide, wide, wide, wide, slim, slim],
      out_specs=slim,
      out_shape=jax.ShapeDtypeStruct((ROWS, GROUPS), jnp.float32),
  )(au2, bu2, am2, bm2, bug2, bmg2)


def kernel(x, u_table, m_table, Bu, Bm):
  users = x[:, 0].astype(jnp.int32)
  movies = x[:, 1].astype(jnp.int32)

  au, bu_, am, bm_, bug, bmg = _sc_gather(
      users, movies, u_table.reshape(-1), m_table.reshape(-1), Bu, Bm)

  out2 = _tc_math(
      au.reshape(ROWS, LANES), bu_.reshape(ROWS, LANES),
      am.reshape(ROWS, LANES), bm_.reshape(ROWS, LANES),
      bug.reshape(ROWS, GROUPS), bmg.reshape(ROWS, GROUPS))
  return out2.reshape(B)


# speedup vs baseline: 1.4915x; 1.4915x over previous
import functools

import jax
import jax.numpy as jnp
from jax import lax
from jax.experimental import pallas as pl
from jax.experimental.pallas import tpu as pltpu
from jax.experimental.pallas import tpu_sc as plsc

B = 16384
D = 16
NC = 2
NS = 16
NW = NC * NS
BPW = B // NW
CHUNK = 128
NCHUNK = BPW // CHUNK

LANES = 128
ROWS = B * D // LANES
GRID = 8
BLK = ROWS // GRID
GROUPS = LANES // D


def _sc_gather(users, movies, tab_u, tab_m, bu_t, bm_t):
  mesh = plsc.VectorSubcoreMesh(core_axis_name="c", subcore_axis_name="s")
  f32 = jnp.float32

  @functools.partial(
      pl.kernel,
      mesh=mesh,
      out_type=[
          jax.ShapeDtypeStruct((B * D,), f32),
          jax.ShapeDtypeStruct((B * D,), f32),
          jax.ShapeDtypeStruct((B * D,), f32),
          jax.ShapeDtypeStruct((B * D,), f32),
          jax.ShapeDtypeStruct((B,), f32),
          jax.ShapeDtypeStruct((B,), f32),
      ],
      scratch_types=[
          pltpu.VMEM((BPW,), jnp.int32),
          pltpu.VMEM((BPW,), jnp.int32),
          pltpu.VMEM((BPW * D,), f32),
          pltpu.VMEM((BPW * D,), f32),
          pltpu.VMEM((BPW * D,), f32),
          pltpu.VMEM((BPW * D,), f32),
          pltpu.VMEM((BPW,), f32),
          pltpu.VMEM((BPW,), f32),
          pltpu.SemaphoreType.DMA,
          pltpu.SemaphoreType.DMA,
      ],
  )
  def k(us_h, mv_h, tu_h, tm_h, bu_h, bm_h,
        au_o, buo_o, am_o, bmo_o, bug_o, bmg_o,
        us_v, mv_v, au_v, buv_v, am_v, bmv_v,
        bus_v, bms_v, isem, sem):
    wid = lax.axis_index("s") * NC + lax.axis_index("c")
    base = wid * BPW
    sl = pl.ds(base, BPW)

    hs = [pltpu.async_copy(us_h.at[sl], us_v, isem),
          pltpu.async_copy(mv_h.at[sl], mv_v, isem)]
    for h in hs:
      h.wait()

    bias_hs = []
    for tab, idx_v, dst_v in ((bu_h, us_v, bus_v), (bm_h, mv_v, bms_v)):
      for c in range(NCHUNK):
        cs = pl.ds(c * CHUNK, CHUNK)
        bias_hs.append(
            pltpu.async_copy(tab.at[idx_v.at[cs]], dst_v.at[cs], isem))

    def body(g, _):
      gsl = pl.ds(pl.multiple_of(g * 16, 16), 16)
      uvec = us_v[gsl]
      mvec = mv_v[gsl]
      for j in range(16):
        u = uvec[j]
        m = mvec[j]
        o = pl.multiple_of(g * (16 * D) + j * D, D)
        pltpu.async_copy(tu_h.at[u, pl.ds(0, D)], au_v.at[pl.ds(o, D)], sem)
        pltpu.async_copy(tu_h.at[u, pl.ds(D, D)], buv_v.at[pl.ds(o, D)], sem)
        pltpu.async_copy(tm_h.at[m, pl.ds(0, D)], am_v.at[pl.ds(o, D)], sem)
        pltpu.async_copy(tm_h.at[m, pl.ds(D, D)], bmv_v.at[pl.ds(o, D)], sem)
      return ()
    lax.fori_loop(0, BPW // 16, body, ())

    for buf in (au_v, buv_v, am_v, bmv_v):
      pltpu.make_async_copy(au_o.at[pl.ds(0, BPW * D)], buf, sem).wait()
    for h in bias_hs:
      h.wait()

    fsl = pl.ds(base * D, BPW * D)
    outs = [(au_v, au_o.at[fsl]), (buv_v, buo_o.at[fsl]),
            (am_v, am_o.at[fsl]), (bmv_v, bmo_o.at[fsl]),
            (bus_v, bug_o.at[sl]), (bms_v, bmg_o.at[sl])]
    hs = [pltpu.async_copy(src, dst, isem) for src, dst in outs]
    for h in hs:
      h.wait()

  return k(users, movies, tab_u, tab_m, bu_t, bm_t)


_HL2PI = 0.9189385332046727


def _stirling(z):
  r = 1.0 / z
  w = r * r
  series = r * (8.333333333333333e-2
                + w * (-2.777777777777778e-3 + w * 7.936507936507937e-4))
  return (z - 0.5) * jnp.log(z) - z + _HL2PI + series


def _lgamma_1(x):
  return _stirling(x + 3.0) - jnp.log(x * (x + 1.0) * (x + 2.0))


def _lgamma_2(s):
  return _stirling(s + 2.0) - jnp.log(s * (s + 1.0))


def _dg_series(z):
  r = 1.0 / z
  w = r * r
  return (jnp.log(z) - 0.5 * r
          - w * (8.333333333333333e-2
                 + w * (-8.333333333333333e-3 + w * 3.968253968253968e-3)))


def _digamma_1(x):
  num = 3.0 * x * x + 6.0 * x + 2.0
  den = x * (x + 1.0) * (x + 2.0)
  return _dg_series(x + 3.0) - num / den


def _digamma_2(s):
  return _dg_series(s + 2.0) - (2.0 * s + 1.0) / (s * (s + 1.0))


def _math_body(au_ref, bu_ref, am_ref, bm_ref, bug_ref, bmg_ref, out_ref):
  def fix(v):
    v = jnp.where(jnp.isnan(v), 0.05, v)
    return jnp.clip(v + 1.0, 1.0, 100.0)

  a1 = fix(au_ref[...])
  b1 = fix(bu_ref[...])
  a2 = fix(am_ref[...])
  b2 = fix(bm_ref[...])
  s1 = a1 + b1
  s2 = a2 + b2

  lnB1 = _lgamma_1(a1) + _lgamma_1(b1) - _lgamma_2(s1)
  lnB2 = _lgamma_1(a2) + _lgamma_1(b2) - _lgamma_2(s2)
  kl = (lnB2 - lnB1
        + (a1 - a2) * _digamma_1(a1)
        + (b1 - b2) * _digamma_1(b1)
        + (a2 - a1 + b2 - b1) * _digamma_2(s1))

  t = jnp.arctan2(jnp.abs(kl), 1.0) * (2.0 / jnp.pi)

  ri = lax.broadcasted_iota(jnp.int32, (LANES, GROUPS), 0)
  ci = lax.broadcasted_iota(jnp.int32, (LANES, GROUPS), 1)
  sel = jnp.where((ri // D) == ci, 1.0, 0.0).astype(jnp.float32)
  dist = jnp.dot(t, sel, preferred_element_type=jnp.float32)

  out_ref[...] = bug_ref[...] + bmg_ref[...] - dist


def _tc_math(au2, bu2, am2, bm2, bug2, bmg2):
  wide = pl.BlockSpec((BLK, LANES), lambda i: (i, 0))
  slim = pl.BlockSpec((BLK, GROUPS), lambda i: (i, 0))
  return pl.pallas_call(
      _math_body,
      grid=(GRID,),
      in_specs=[wide, wide, wide, wide, slim, slim],
      out_specs=slim,
      out_shape=jax.ShapeDtypeStruct((ROWS, GROUPS), jnp.float32),
  )(au2, bu2, am2, bm2, bug2, bmg2)


def kernel(x, u_table, m_table, Bu, Bm):
  users = x[:, 0].astype(jnp.int32)
  movies = x[:, 1].astype(jnp.int32)

  au, bu_, am, bm_, bug, bmg = _sc_gather(
      users, movies, u_table, m_table, Bu, Bm)

  out2 = _tc_math(
      au.reshape(ROWS, LANES), bu_.reshape(ROWS, LANES),
      am.reshape(ROWS, LANES), bm_.reshape(ROWS, LANES),
      bug.reshape(ROWS, GROUPS), bmg.reshape(ROWS, GROUPS))
  return out2.reshape(B)
